# baseline (speedup 1.0000x reference)
import jax
import jax.numpy as jnp
from jax import lax
from jax.experimental import pallas as pl
from jax.experimental.pallas import tpu as pltpu
from jax.experimental.pallas import tpu_sc as plsc

N = 10000
E = 320000
D = 128
BN_EPS = 1e-5

NC = 2
NS = 16
NW = NC * NS

K = 128
NCH = 82
EPT = NCH * K
E_PAD = NW * EPT
N_PAD = 10112
RPS = N_PAD // NS


def _sc_segment_sum_body(x_hbm, zeros_hbm, sd_hbm, out_hbm,
                         sd0, sd1, sd2, r0, r1, r2, acc_sh,
                         g0, g1, g2, s0, s1, s2):
    sd = (sd0, sd1, sd2)
    rows = (r0, r1, r2)
    gsem = (g0, g1, g2)
    ssem = (s0, s1, s2)
    c = lax.axis_index("c")
    s = lax.axis_index("s")
    wid = s * NC + c

    def fetch(ch, b):
        pltpu.sync_copy(sd_hbm.at[wid, ch], sd[b])

    def gather(b):
        pltpu.async_copy(x_hbm.at[sd[b].at[0]], rows[b], gsem[b])

    def gather_wait(b):
        pltpu.make_async_copy(x_hbm.at[sd[b].at[0]], rows[b], gsem[b]).wait()

    def scatter(b):
        pltpu.async_copy(rows[b], acc_sh.at[sd[b].at[1]], ssem[b], add=True)

    def scatter_wait(b):
        pltpu.make_async_copy(rows[b], acc_sh.at[sd[b].at[1]], ssem[b]).wait()

    pltpu.sync_copy(zeros_hbm.at[pl.ds(s * RPS, RPS)],
                    acc_sh.at[pl.ds(s * RPS, RPS)])
    plsc.subcore_barrier()

    def chunk(i, carry):
        fetch(i, 0)
        pltpu.sync_copy(x_hbm.at[sd0.at[0]], r0)
        pltpu.sync_copy(r0, acc_sh.at[sd0.at[1]], add=True)
        return carry

    lax.fori_loop(0, NCH, chunk, 0)
    plsc.subcore_barrier()

    pltpu.sync_copy(acc_sh.at[pl.ds(s * RPS, RPS)],
                    out_hbm.at[c, pl.ds(s * RPS, RPS)])


_sc_segment_sum = pl.kernel(
    _sc_segment_sum_body,
    out_type=jax.ShapeDtypeStruct((NC, N_PAD, D), jnp.float32),
    mesh=plsc.VectorSubcoreMesh(core_axis_name="c", subcore_axis_name="s",
                                num_cores=NC, num_subcores=NS),
    scratch_types=(
        [pltpu.VMEM((2, K), jnp.int32)] * 3
        + [pltpu.VMEM((K, D), jnp.float32)] * 3
        + [pltpu.VMEM_SHARED((N_PAD, D), jnp.float32)]
        + [pltpu.SemaphoreType.DMA] * 6
    ),
)


def _dense_body(x_ref, agg_ref, w_ref, b_ref, g_ref, be_ref, o_ref):
    h = x_ref[...] + agg_ref[0, :N, :] + agg_ref[1, :N, :]
    z = jnp.dot(h, w_ref[...], preferred_element_type=jnp.float32) + b_ref[...]
    mu = jnp.mean(z, axis=0, keepdims=True)
    zc = z - mu
    var = jnp.mean(zc * zc, axis=0, keepdims=True)
    y = g_ref[...] * zc * lax.rsqrt(var + BN_EPS) + be_ref[...]
    o_ref[...] = jnp.maximum(y, 0.0)


_dense_layer = pl.pallas_call(
    _dense_body,
    out_shape=jax.ShapeDtypeStruct((N, D), jnp.float32),
)


def kernel(g, features, W1, b1, gamma1, beta1, W2, b2, gamma2, beta2):
    src = g[0]
    dst = g[1]
    pad = E_PAD - E
    srcp = jnp.concatenate([src, jnp.zeros((pad,), jnp.int32)]).reshape(NW, NCH, K)
    dstp = jnp.concatenate([dst, jnp.full((pad,), N, jnp.int32)]).reshape(NW, NCH, K)
    sd = jnp.stack([srcp, dstp], axis=2)
    zeros = jnp.zeros((N_PAD, D), jnp.float32)

    b1r, g1r, be1r = b1.reshape(1, D), gamma1.reshape(1, D), beta1.reshape(1, D)
    b2r, g2r, be2r = b2.reshape(1, D), gamma2.reshape(1, D), beta2.reshape(1, D)

    agg1 = _sc_segment_sum(features, zeros, sd)
    y1 = _dense_layer(features, agg1, W1, b1r, g1r, be1r)
    agg2 = _sc_segment_sum(y1, zeros, sd)
    y2 = _dense_layer(y1, agg2, W2, b2r, g2r, be2r)
    return y2


# speedup vs baseline: 1.9551x; 1.9551x over previous
import jax
import jax.numpy as jnp
from jax import lax
from jax.experimental import pallas as pl
from jax.experimental.pallas import tpu as pltpu
from jax.experimental.pallas import tpu_sc as plsc

N = 10000
E = 320000
D = 128
BN_EPS = 1e-5

NC = 2
NS = 16
NW = NC * NS

K = 128
NCH = 79
EPT = NCH * K
E_PAD = NW * EPT
N_PAD = 10112
RPS = N_PAD // NS


def _sc_segment_sum_body(x_hbm, zeros_hbm, src_hbm, dst_hbm, out_hbm,
                         src_v, dst_v, rows_v, acc_sh, sem):
    c = lax.axis_index("c")
    s = lax.axis_index("s")
    wid = s * NC + c

    pltpu.sync_copy(zeros_hbm.at[pl.ds(s * RPS, RPS)],
                    acc_sh.at[pl.ds(s * RPS, RPS)])
    plsc.subcore_barrier()

    def chunk(i, carry):
        pltpu.sync_copy(src_hbm.at[wid, i], src_v)
        pltpu.sync_copy(dst_hbm.at[wid, i], dst_v)
        pltpu.async_copy(x_hbm.at[src_v], rows_v, sem).wait()
        pltpu.sync_copy(rows_v, acc_sh.at[dst_v], add=True)
        return carry

    lax.fori_loop(0, NCH, chunk, 0)
    plsc.subcore_barrier()

    pltpu.sync_copy(acc_sh.at[pl.ds(s * RPS, RPS)],
                    out_hbm.at[c, pl.ds(s * RPS, RPS)])


_sc_segment_sum = pl.kernel(
    _sc_segment_sum_body,
    out_type=jax.ShapeDtypeStruct((NC, N_PAD, D), jnp.float32),
    mesh=plsc.VectorSubcoreMesh(core_axis_name="c", subcore_axis_name="s",
                                num_cores=NC, num_subcores=NS),
    scratch_types=[
        pltpu.VMEM((K,), jnp.int32),
        pltpu.VMEM((K,), jnp.int32),
        pltpu.VMEM((K, D), jnp.float32),
        pltpu.VMEM_SHARED((N_PAD, D), jnp.float32),
        pltpu.SemaphoreType.DMA,
    ],
)


def _dense_body(x_ref, agg_ref, w_ref, b_ref, g_ref, be_ref, o_ref):
    h = x_ref[...] + agg_ref[0, :N, :] + agg_ref[1, :N, :]
    z = jnp.dot(h, w_ref[...], preferred_element_type=jnp.float32) + b_ref[...]
    mu = jnp.mean(z, axis=0, keepdims=True)
    zc = z - mu
    var = jnp.mean(zc * zc, axis=0, keepdims=True)
    y = g_ref[...] * zc * lax.rsqrt(var + BN_EPS) + be_ref[...]
    o_ref[...] = jnp.maximum(y, 0.0)


_dense_layer = pl.pallas_call(
    _dense_body,
    out_shape=jax.ShapeDtypeStruct((N, D), jnp.float32),
)


def kernel(g, features, W1, b1, gamma1, beta1, W2, b2, gamma2, beta2):
    src = g[0]
    dst = g[1]
    pad = E_PAD - E
    srcp = jnp.concatenate([src, jnp.zeros((pad,), jnp.int32)]).reshape(NW, NCH, K)
    dstp = jnp.concatenate([dst, jnp.full((pad,), N, jnp.int32)]).reshape(NW, NCH, K)
    zeros = jnp.zeros((N_PAD, D), jnp.float32)

    b1r, g1r, be1r = b1.reshape(1, D), gamma1.reshape(1, D), beta1.reshape(1, D)
    b2r, g2r, be2r = b2.reshape(1, D), gamma2.reshape(1, D), beta2.reshape(1, D)

    agg1 = _sc_segment_sum(features, zeros, srcp, dstp)
    y1 = _dense_layer(features, agg1, W1, b1r, g1r, be1r)
    agg2 = _sc_segment_sum(y1, zeros, srcp, dstp)
    y2 = _dense_layer(y1, agg2, W2, b2r, g2r, be2r)
    return y2
